# Initial kernel scaffold; baseline (speedup 1.0000x reference)
import jax
import jax.numpy as jnp
from jax.experimental import pallas as pl


def kernel(obs, preference, h_w1, h_b1, h_ln1_g, h_ln1_b, h_w2, h_b2, h_ln2_g, h_ln2_b, p_w, p_b, p_ln_g, p_ln_b, s_w1, s_b1, s_ln1_g, s_ln1_b, s_w2, s_b2, s_ln2_g, s_ln2_b, qd_w, qd_b, qe_w, qe_b):
    raise NotImplementedError("write your pallas kernel here")


# speedup vs baseline: 1.1895x; 1.1895x over previous
import jax
import jax.numpy as jnp
from jax.experimental import pallas as pl
from jax.experimental.pallas import tpu as pltpu

_NUM_BINS = 10
_HIST_LO = 0.0
_HIST_HI = 10.0
_LN_EPS = 1e-5
_START = 68
_NSRV = 64


def _ln(x, g, b):
    mu = jnp.mean(x, axis=-1, keepdims=True)
    xc = x - mu
    var = jnp.mean(xc * xc, axis=-1, keepdims=True)
    return xc * jax.lax.rsqrt(var + _LN_EPS) * g + b


def _relu(x):
    return jnp.maximum(x, 0.0)


def _body(obs_ref, pref_ref,
          hw1_ref, hb1_ref, hg1_ref, hbt1_ref,
          hw2_ref, hb2_ref, hg2_ref, hbt2_ref,
          pw_ref, pb_ref, pg_ref, pbt_ref,
          wobs_ref, wh_ref, wp_ref,
          sb1_ref, sg1_ref, sbt1_ref,
          sw2_ref, sb2_ref, sg2_ref, sbt2_ref,
          wq_ref, bq_ref,
          o_ref):
    f32 = jnp.float32
    obs = obs_ref[...]

    w = obs[:, _START:_START + _NSRV]
    e = jnp.floor(w)
    valid = (w >= _HIST_LO) & (w <= _HIST_HI)
    ef = jnp.where(valid, jnp.clip(e, 0.0, float(_NUM_BINS - 1)), -1.0)
    total = jnp.sum(jnp.where(valid, 1.0, 0.0), axis=1, keepdims=True)
    acc = None
    for k in range(_NUM_BINS):
        ck = jnp.sum(jnp.where(ef == float(k), 1.0, 0.0), axis=1,
                     keepdims=True)
        term = ck * hw1_ref[k:k + 1, :]
        acc = term if acc is None else acc + term
    rcp = 1.0 / (total + 1e-8)
    h1 = _ln(_relu(acc * rcp + hb1_ref[...]), hg1_ref[...], hbt1_ref[...])
    h2_pre = jnp.dot(h1, hw2_ref[...], preferred_element_type=f32)
    h2 = _ln(_relu(h2_pre + hb2_ref[...]), hg2_ref[...], hbt2_ref[...])

    pref = pref_ref[...]
    p_pre = (pref[:, 0:1] * pw_ref[0:1, :]
             + pref[:, 1:2] * pw_ref[1:2, :] + pb_ref[...])
    p = _ln(_relu(p_pre), pg_ref[...], pbt_ref[...])

    s1_pre = (jnp.dot(obs, wobs_ref[...], preferred_element_type=f32)
              + jnp.dot(h2, wh_ref[...], preferred_element_type=f32)
              + jnp.dot(p, wp_ref[...], preferred_element_type=f32)
              + sb1_ref[...])
    s1 = _ln(_relu(s1_pre), sg1_ref[...], sbt1_ref[...])
    s2_pre = jnp.dot(s1, sw2_ref[...], preferred_element_type=f32)
    s2 = _ln(_relu(s2_pre + sb2_ref[...]), sg2_ref[...], sbt2_ref[...])

    o_ref[...] = jnp.dot(s2, wq_ref[...], preferred_element_type=f32) + bq_ref[...]


def kernel(obs, preference,
           h_w1, h_b1, h_ln1_g, h_ln1_b, h_w2, h_b2, h_ln2_g, h_ln2_b,
           p_w, p_b, p_ln_g, p_ln_b,
           s_w1, s_b1, s_ln1_g, s_ln1_b, s_w2, s_b2, s_ln2_g, s_ln2_b,
           qd_w, qd_b, qe_w, qe_b):
    B, OBS = obs.shape
    ACT = qd_w.shape[1]
    blk = min(1024, B)

    w_obs = jnp.concatenate(
        [s_w1[:_START],
         jnp.zeros((_NSRV, s_w1.shape[1]), s_w1.dtype),
         s_w1[_START:OBS - _NSRV]], axis=0)
    w_h = s_w1[OBS - _NSRV:OBS - _NSRV + 128]
    w_p = s_w1[OBS - _NSRV + 128:]
    w_q = jnp.stack([qd_w, qe_w], axis=-1).reshape(qd_w.shape[0], 2 * ACT)
    b_q = jnp.stack([qd_b, qe_b], axis=-1).reshape(1, 2 * ACT)

    def row(v):
        return v.reshape(1, -1)

    def wspec(shape):
        return pl.BlockSpec(shape, lambda i: (0, 0))

    ins = (obs, preference,
           h_w1, row(h_b1), row(h_ln1_g), row(h_ln1_b),
           h_w2, row(h_b2), row(h_ln2_g), row(h_ln2_b),
           p_w, row(p_b), row(p_ln_g), row(p_ln_b),
           w_obs, w_h, w_p,
           row(s_b1), row(s_ln1_g), row(s_ln1_b),
           s_w2, row(s_b2), row(s_ln2_g), row(s_ln2_b),
           w_q, b_q)

    in_specs = [pl.BlockSpec((blk, OBS), lambda i: (i, 0)),
                pl.BlockSpec((blk, 2), lambda i: (i, 0))]
    in_specs += [wspec(x.shape) for x in ins[2:]]

    out = pl.pallas_call(
        _body,
        grid=(B // blk,),
        in_specs=in_specs,
        out_specs=pl.BlockSpec((blk, 2 * ACT), lambda i: (i, 0)),
        out_shape=jax.ShapeDtypeStruct((B, 2 * ACT), jnp.float32),
        compiler_params=pltpu.CompilerParams(
            dimension_semantics=("parallel",),
        ),
        name="critic_fused",
    )(*ins)
    return out.reshape(B, ACT, 2)
